# baseline (speedup 1.0000x reference)
import functools
import math

import jax
import jax.numpy as jnp
from jax import lax
from jax.experimental import pallas as pl
from jax.experimental.pallas import tpu as pltpu
from jax.experimental.pallas import tpu_sc as plsc

D_MODEL = 64
SCALE = math.sqrt(D_MODEL)

_NC = 2
_NS = 16
_NW = _NC * _NS

_CH = 512


def _emb_body(b_per_w, n_chunks, x_hbm, table_hbm, out_hbm, idx_v, rows_v, sem):
    wid = lax.axis_index("s") * _NC + lax.axis_index("c")
    base = wid * b_per_w
    pltpu.sync_copy(x_hbm.at[pl.ds(base, b_per_w)], idx_v)

    def chunk_body(g, carry):
        off = pl.multiple_of(g * _CH, 8)
        idx_slice = idx_v.at[pl.ds(off, _CH)]
        pltpu.async_copy(table_hbm.at[idx_slice], rows_v, sem).wait()

        def scale_row(r, c):
            for k in range(D_MODEL // 16):
                rows_v[r, pl.ds(k * 16, 16)] = rows_v[r, pl.ds(k * 16, 16)] * SCALE
            return c

        lax.fori_loop(0, _CH, scale_row, 0)
        out_off = pl.multiple_of(base + off, 8)
        pltpu.sync_copy(rows_v, out_hbm.at[pl.ds(out_off, _CH)])
        return carry

    lax.fori_loop(0, n_chunks, chunk_body, 0)


@functools.partial(jax.jit, static_argnames=())
def kernel(x, table):
    B = x.shape[0] * x.shape[1]
    assert B % (_NW * _CH) == 0
    b_per_w = B // _NW
    n_chunks = b_per_w // _CH

    xf = x.reshape(-1).astype(jnp.int32)
    mesh = plsc.VectorSubcoreMesh(core_axis_name="c", subcore_axis_name="s")
    emb = pl.kernel(
        functools.partial(_emb_body, b_per_w, n_chunks),
        mesh=mesh,
        out_type=jax.ShapeDtypeStruct((B, D_MODEL), jnp.float32),
        scratch_types=[
            pltpu.VMEM((b_per_w,), jnp.int32),
            pltpu.VMEM((_CH, D_MODEL), jnp.float32),
            pltpu.SemaphoreType.DMA,
        ],
        compiler_params=pltpu.CompilerParams(use_tc_tiling_on_sc=False),
    )
    out = emb(xf, table)
    return out.reshape(x.shape[0], x.shape[1], D_MODEL)
